# Initial kernel scaffold; baseline (speedup 1.0000x reference)
import jax
import jax.numpy as jnp
from jax.experimental import pallas as pl


def kernel(x, edge_index, W, b):
    raise NotImplementedError("write your pallas kernel here")


# speedup vs baseline: 7.5660x; 7.5660x over previous
import functools

import jax
import jax.numpy as jnp
from jax import lax
from jax.experimental import pallas as pl
from jax.experimental.pallas import tpu as pltpu
from jax.experimental.pallas import tpu_sc as plsc

N = 10000
E = 320000
D = 128

NC = 2
NS = 16
NW = NC * NS

CHUNK = 80
EPT = E // NW
NCHUNK = EPT // CHUNK
ROWS_PER_TILE = 632
NPAD = ROWS_PER_TILE * NS


def _sc_aggregate_body(x_hbm, src_hbm, dst_hbm, zeros_hbm, agg_hbm,
                       src_v, dst_v, rows_v, agg_sh, sem):
    cid = lax.axis_index("c")
    sid = lax.axis_index("s")
    wid = cid * NS + sid

    r0 = sid * ROWS_PER_TILE
    pltpu.sync_copy(zeros_hbm.at[pl.ds(r0, ROWS_PER_TILE)],
                    agg_sh.at[pl.ds(r0, ROWS_PER_TILE)])

    pltpu.sync_copy(src_hbm.at[wid], src_v)
    pltpu.sync_copy(dst_hbm.at[wid], dst_v)

    plsc.subcore_barrier()

    def body(j, _):
        pltpu.async_copy(x_hbm.at[src_v.at[j]], rows_v, sem).wait()
        pltpu.sync_copy(rows_v, agg_sh.at[dst_v.at[j]], add=True)
        return _

    lax.fori_loop(0, NCHUNK, body, None)

    plsc.subcore_barrier()

    pltpu.sync_copy(agg_sh.at[pl.ds(r0, ROWS_PER_TILE)],
                    agg_hbm.at[cid, pl.ds(r0, ROWS_PER_TILE)])


def _sc_aggregate(x, src3, dst3, zeros):
    mesh = plsc.VectorSubcoreMesh(core_axis_name="c", subcore_axis_name="s")
    fn = functools.partial(
        pl.kernel,
        out_type=jax.ShapeDtypeStruct((NC, NPAD, D), jnp.float32),
        mesh=mesh,
        scratch_types=[
            pltpu.VMEM((NCHUNK, CHUNK), jnp.int32),
            pltpu.VMEM((NCHUNK, CHUNK), jnp.int32),
            pltpu.VMEM((CHUNK, D), jnp.float32),
            pltpu.VMEM_SHARED((NPAD, D), jnp.float32),
            pltpu.SemaphoreType.DMA,
        ],
    )(_sc_aggregate_body)
    return fn(x, src3, dst3, zeros)


def _tc_linear_body(x_ref, agg_ref, wt_ref, b_ref, out_ref):
    h = x_ref[...] + agg_ref[0] + agg_ref[1]
    h = jnp.dot(h, wt_ref[...], preferred_element_type=jnp.float32)
    out_ref[...] = jnp.maximum(h + b_ref[...], 0.0)


def _tc_linear(x, agg, wt, b2):
    bm = 1000
    grid = (N // bm,)
    return pl.pallas_call(
        _tc_linear_body,
        grid=grid,
        in_specs=[
            pl.BlockSpec((bm, D), lambda i: (i, 0)),
            pl.BlockSpec((NC, bm, D), lambda i: (0, i, 0)),
            pl.BlockSpec((D, D), lambda i: (0, 0)),
            pl.BlockSpec((1, D), lambda i: (0, 0)),
        ],
        out_specs=pl.BlockSpec((bm, D), lambda i: (i, 0)),
        out_shape=jax.ShapeDtypeStruct((N, D), jnp.float32),
    )(x, agg, wt, b2)


def kernel(x, edge_index, W, b):
    src3 = edge_index[0].astype(jnp.int32).reshape(NW, NCHUNK, CHUNK)
    dst3 = edge_index[1].astype(jnp.int32).reshape(NW, NCHUNK, CHUNK)
    zeros = jnp.zeros((NPAD, D), jnp.float32)
    agg = _sc_aggregate(x, src3, dst3, zeros)
    return _tc_linear(x, agg, W.T, b.reshape(1, D))


# speedup vs baseline: 11.0203x; 1.4566x over previous
import functools

import jax
import jax.numpy as jnp
from jax import lax
from jax.experimental import pallas as pl
from jax.experimental.pallas import tpu as pltpu
from jax.experimental.pallas import tpu_sc as plsc

N = 10000
E = 320000
D = 128

NC = 2
NS = 16
NW = NC * NS

CHUNK = 80
EPT = E // NW
NCHUNK = EPT // CHUNK
BCH = 25
NBLK = NCHUNK // BCH
ROWS_PER_TILE = 632
NPAD = ROWS_PER_TILE * NS


def _sc_aggregate_body(x_hbm, src_hbm, dst_hbm, zeros_hbm, agg_hbm,
                       src_v, dst_v, rows0, rows1, agg_sh, sem0, sem1):
    cid = lax.axis_index("c")
    sid = lax.axis_index("s")
    wid = cid * NS + sid

    r0 = sid * ROWS_PER_TILE
    pltpu.sync_copy(zeros_hbm.at[pl.ds(r0, ROWS_PER_TILE)],
                    agg_sh.at[pl.ds(r0, ROWS_PER_TILE)])

    plsc.subcore_barrier()

    for blk in range(NBLK):
        pltpu.sync_copy(src_hbm.at[wid, blk], src_v)
        pltpu.sync_copy(dst_hbm.at[wid, blk], dst_v)

        pltpu.async_copy(x_hbm.at[src_v.at[0]], rows0, sem0)

        def body(i, _):
            j0 = 2 * i
            pltpu.async_copy(x_hbm.at[src_v.at[j0 + 1]], rows1, sem1)
            pltpu.make_async_copy(x_hbm.at[src_v.at[j0]], rows0, sem0).wait()
            pltpu.sync_copy(rows0, agg_sh.at[dst_v.at[j0]], add=True)
            pltpu.async_copy(x_hbm.at[src_v.at[j0 + 2]], rows0, sem0)
            pltpu.make_async_copy(x_hbm.at[src_v.at[j0 + 1]], rows1,
                                  sem1).wait()
            pltpu.sync_copy(rows1, agg_sh.at[dst_v.at[j0 + 1]], add=True)
            return _

        lax.fori_loop(0, (BCH - 1) // 2, body, None)
        pltpu.make_async_copy(x_hbm.at[src_v.at[BCH - 1]], rows0, sem0).wait()
        pltpu.sync_copy(rows0, agg_sh.at[dst_v.at[BCH - 1]], add=True)

    plsc.subcore_barrier()

    pltpu.sync_copy(agg_sh.at[pl.ds(r0, ROWS_PER_TILE)],
                    agg_hbm.at[cid, pl.ds(r0, ROWS_PER_TILE)])


def _sc_aggregate(x, src3, dst3, zeros):
    mesh = plsc.VectorSubcoreMesh(core_axis_name="c", subcore_axis_name="s")
    fn = functools.partial(
        pl.kernel,
        out_type=jax.ShapeDtypeStruct((NC, NPAD, D), jnp.float32),
        mesh=mesh,
        scratch_types=[
            pltpu.VMEM((BCH, CHUNK), jnp.int32),
            pltpu.VMEM((BCH, CHUNK), jnp.int32),
            pltpu.VMEM((CHUNK, D), jnp.float32),
            pltpu.VMEM((CHUNK, D), jnp.float32),
            pltpu.VMEM_SHARED((NPAD, D), jnp.float32),
            pltpu.SemaphoreType.DMA,
            pltpu.SemaphoreType.DMA,
        ],
    )(_sc_aggregate_body)
    return fn(x, src3, dst3, zeros)


def _tc_linear_body(x_ref, agg_ref, wt_ref, b_ref, out_ref):
    h = x_ref[...] + agg_ref[0] + agg_ref[1]
    h = jnp.dot(h, wt_ref[...], preferred_element_type=jnp.float32)
    out_ref[...] = jnp.maximum(h + b_ref[...], 0.0)


def _tc_linear(x, agg, wt, b2):
    bm = 1000
    grid = (N // bm,)
    return pl.pallas_call(
        _tc_linear_body,
        grid=grid,
        in_specs=[
            pl.BlockSpec((bm, D), lambda i: (i, 0)),
            pl.BlockSpec((NC, bm, D), lambda i: (0, i, 0)),
            pl.BlockSpec((D, D), lambda i: (0, 0)),
            pl.BlockSpec((1, D), lambda i: (0, 0)),
        ],
        out_specs=pl.BlockSpec((bm, D), lambda i: (i, 0)),
        out_shape=jax.ShapeDtypeStruct((N, D), jnp.float32),
    )(x, agg, wt, b2)


def kernel(x, edge_index, W, b):
    src3 = edge_index[0].astype(jnp.int32).reshape(NW, NBLK, BCH, CHUNK)
    dst3 = edge_index[1].astype(jnp.int32).reshape(NW, NBLK, BCH, CHUNK)
    zeros = jnp.zeros((NPAD, D), jnp.float32)
    agg = _sc_aggregate(x, src3, dst3, zeros)
    return _tc_linear(x, agg, W.T, b.reshape(1, D))


# speedup vs baseline: 11.2983x; 1.0252x over previous
import functools

import jax
import jax.numpy as jnp
from jax import lax
from jax.experimental import pallas as pl
from jax.experimental.pallas import tpu as pltpu
from jax.experimental.pallas import tpu_sc as plsc

N = 10000
E = 320000
D = 128

NC = 2
NS = 16
NW = NC * NS

CHUNK = 50
EPT = E // NW
NCHUNK = EPT // CHUNK
BCH = 20
NBLK = NCHUNK // BCH
ZB = 48
ROWS_PER_TILE = 632
NPAD = ROWS_PER_TILE * NS


def _sc_aggregate_body(x_hbm, src_hbm, dst_hbm, zeros_hbm, agg_hbm,
                       src_v, dst_v, rows0, rows1, rows2, rows3, agg_sh,
                       sem0, sem1, sem2, sem3):
    cid = lax.axis_index("c")
    sid = lax.axis_index("s")
    wid = cid * NS + sid

    r0 = sid * ROWS_PER_TILE
    pltpu.sync_copy(zeros_hbm, rows0.at[pl.ds(0, ZB)])

    @pl.when(cid == 0)
    def _():
        @pl.when(sid < NS - 1)
        def _():
            pltpu.sync_copy(x_hbm.at[pl.ds(r0, ROWS_PER_TILE)],
                            agg_sh.at[pl.ds(r0, ROWS_PER_TILE)])

        @pl.when(sid == NS - 1)
        def _():
            pltpu.sync_copy(x_hbm.at[pl.ds(r0, N - r0)],
                            agg_sh.at[pl.ds(r0, N - r0)])
            pltpu.sync_copy(rows0.at[pl.ds(0, ZB)],
                            agg_sh.at[pl.ds(N, ZB)])
            pltpu.sync_copy(rows0.at[pl.ds(0, ZB)],
                            agg_sh.at[pl.ds(N + ZB, ZB)])
            pltpu.sync_copy(rows0.at[pl.ds(0, NPAD - N - 2 * ZB)],
                            agg_sh.at[pl.ds(N + 2 * ZB, NPAD - N - 2 * ZB)])

    @pl.when(cid == 1)
    def _():
        for t in range(ROWS_PER_TILE // ZB):
            pltpu.sync_copy(rows0.at[pl.ds(0, ZB)],
                            agg_sh.at[pl.ds(r0 + t * ZB, ZB)])
        rem = ROWS_PER_TILE % ZB
        pltpu.sync_copy(
            rows0.at[pl.ds(0, rem)],
            agg_sh.at[pl.ds(r0 + ROWS_PER_TILE - rem, rem)])

    plsc.subcore_barrier()

    for blk in range(NBLK):
        pltpu.sync_copy(src_hbm.at[wid, blk], src_v)
        pltpu.sync_copy(dst_hbm.at[wid, blk], dst_v)

        pltpu.async_copy(x_hbm.at[src_v.at[0]], rows0, sem0)
        pltpu.async_copy(x_hbm.at[src_v.at[1]], rows1, sem1)
        pltpu.async_copy(x_hbm.at[src_v.at[2]], rows2, sem2)

        def body(i, _):
            j0 = 4 * i
            pltpu.async_copy(x_hbm.at[src_v.at[j0 + 3]], rows3, sem3)
            pltpu.make_async_copy(x_hbm.at[src_v.at[j0]], rows0, sem0).wait()
            pltpu.sync_copy(rows0, agg_sh.at[dst_v.at[j0]], add=True)
            pltpu.async_copy(x_hbm.at[src_v.at[j0 + 4]], rows0, sem0)
            pltpu.make_async_copy(x_hbm.at[src_v.at[j0 + 1]], rows1,
                                  sem1).wait()
            pltpu.sync_copy(rows1, agg_sh.at[dst_v.at[j0 + 1]], add=True)
            pltpu.async_copy(x_hbm.at[src_v.at[j0 + 5]], rows1, sem1)
            pltpu.make_async_copy(x_hbm.at[src_v.at[j0 + 2]], rows2,
                                  sem2).wait()
            pltpu.sync_copy(rows2, agg_sh.at[dst_v.at[j0 + 2]], add=True)
            pltpu.async_copy(x_hbm.at[src_v.at[j0 + 6]], rows2, sem2)
            pltpu.make_async_copy(x_hbm.at[src_v.at[j0 + 3]], rows3,
                                  sem3).wait()
            pltpu.sync_copy(rows3, agg_sh.at[dst_v.at[j0 + 3]], add=True)
            return _

        lax.fori_loop(0, (BCH - 4) // 4, body, None)

        jlast = BCH - 4
        pltpu.async_copy(x_hbm.at[src_v.at[jlast + 3]], rows3, sem3)
        pltpu.make_async_copy(x_hbm.at[src_v.at[jlast]], rows0, sem0).wait()
        pltpu.sync_copy(rows0, agg_sh.at[dst_v.at[jlast]], add=True)
        pltpu.make_async_copy(x_hbm.at[src_v.at[jlast + 1]], rows1,
                              sem1).wait()
        pltpu.sync_copy(rows1, agg_sh.at[dst_v.at[jlast + 1]], add=True)
        pltpu.make_async_copy(x_hbm.at[src_v.at[jlast + 2]], rows2,
                              sem2).wait()
        pltpu.sync_copy(rows2, agg_sh.at[dst_v.at[jlast + 2]], add=True)
        pltpu.make_async_copy(x_hbm.at[src_v.at[jlast + 3]], rows3,
                              sem3).wait()
        pltpu.sync_copy(rows3, agg_sh.at[dst_v.at[jlast + 3]], add=True)

    plsc.subcore_barrier()

    pltpu.sync_copy(agg_sh.at[pl.ds(r0, ROWS_PER_TILE)],
                    agg_hbm.at[cid, pl.ds(r0, ROWS_PER_TILE)])


def _sc_aggregate(x, src3, dst3, zeros):
    mesh = plsc.VectorSubcoreMesh(core_axis_name="c", subcore_axis_name="s")
    fn = functools.partial(
        pl.kernel,
        out_type=jax.ShapeDtypeStruct((NC, NPAD, D), jnp.float32),
        mesh=mesh,
        scratch_types=[
            pltpu.VMEM((BCH, CHUNK), jnp.int32),
            pltpu.VMEM((BCH, CHUNK), jnp.int32),
            pltpu.VMEM((CHUNK, D), jnp.float32),
            pltpu.VMEM((CHUNK, D), jnp.float32),
            pltpu.VMEM((CHUNK, D), jnp.float32),
            pltpu.VMEM((CHUNK, D), jnp.float32),
            pltpu.VMEM_SHARED((NPAD, D), jnp.float32),
            pltpu.SemaphoreType.DMA,
            pltpu.SemaphoreType.DMA,
            pltpu.SemaphoreType.DMA,
            pltpu.SemaphoreType.DMA,
        ],
    )(_sc_aggregate_body)
    return fn(x, src3, dst3, zeros)


def _tc_linear_body(agg_ref, wt_ref, b_ref, out_ref):
    h = agg_ref[0] + agg_ref[1]
    h = jnp.dot(h, wt_ref[...], preferred_element_type=jnp.float32)
    out_ref[...] = jnp.maximum(h + b_ref[...], 0.0)


def _tc_linear(agg, wt, b2):
    bm = 1000
    grid = (N // bm,)
    return pl.pallas_call(
        _tc_linear_body,
        grid=grid,
        in_specs=[
            pl.BlockSpec((NC, bm, D), lambda i: (0, i, 0)),
            pl.BlockSpec((D, D), lambda i: (0, 0)),
            pl.BlockSpec((1, D), lambda i: (0, 0)),
        ],
        out_specs=pl.BlockSpec((bm, D), lambda i: (i, 0)),
        out_shape=jax.ShapeDtypeStruct((N, D), jnp.float32),
    )(agg, wt, b2)


def kernel(x, edge_index, W, b):
    src3 = edge_index[0].astype(jnp.int32).reshape(NW, NBLK, BCH, CHUNK)
    dst3 = edge_index[1].astype(jnp.int32).reshape(NW, NBLK, BCH, CHUNK)
    zeros = jnp.zeros((ZB, D), jnp.float32)
    agg = _sc_aggregate(x, src3, dst3, zeros)
    return _tc_linear(agg, W.T, b.reshape(1, D))
